# baseline (speedup 1.0000x reference)
import functools

import jax
import jax.numpy as jnp
from jax import lax
from jax.experimental import pallas as pl
from jax.experimental.pallas import tpu as pltpu
from jax.experimental.pallas import tpu_sc as plsc

NA = 10000
NE = 320000
D = 128
NRBF = 16
NBLK = 5
NRI = 3
NRF = 2
LOG2 = 0.6931471805599453

ROWS = 1000
NRT = NA // ROWS
ET = 1280
NT = NE // ET
W = 128

NWORK = 32
EPW = NE // NWORK
CH = 400
NPAIR = 12


def _ssp(x):
    return jax.nn.softplus(x) - LOG2


def _pre_body(x_ref, wi_ref, bi_ref, wj_ref, bj_ref, xi_ref, xj_ref):
    xa = _ssp(x_ref[...])
    xi_ref[...] = _ssp(
        jnp.dot(xa, wi_ref[...], preferred_element_type=jnp.float32) + bi_ref[...])
    xj_ref[...] = _ssp(
        jnp.dot(xa, wj_ref[...], preferred_element_type=jnp.float32) + bj_ref[...])


def _pre(x, wi, bi, wj, bj):
    return pl.pallas_call(
        _pre_body,
        grid=(NRT,),
        in_specs=[
            pl.BlockSpec((ROWS, D), lambda t: (t, 0)),
            pl.BlockSpec((D, D), lambda t: (0, 0)),
            pl.BlockSpec((1, D), lambda t: (0, 0)),
            pl.BlockSpec((D, D), lambda t: (0, 0)),
            pl.BlockSpec((1, D), lambda t: (0, 0)),
        ],
        out_specs=[pl.BlockSpec((ROWS, D), lambda t: (t, 0)),
                   pl.BlockSpec((ROWS, D), lambda t: (t, 0))],
        out_shape=[jax.ShapeDtypeStruct((NA, D), jnp.float32)] * 2,
    )(x, wi, bi.reshape(1, D), wj, bj.reshape(1, D))


def _sc_gather(table, idx):
    mesh = plsc.VectorSubcoreMesh(core_axis_name="c", subcore_axis_name="s")

    @functools.partial(
        pl.kernel, mesh=mesh,
        out_type=jax.ShapeDtypeStruct((NE, D), jnp.float32),
        scratch_types=[
            pltpu.VMEM((EPW,), jnp.int32),
            pltpu.VMEM((CH, D), jnp.float32),
            pltpu.VMEM((CH, D), jnp.float32),
            pltpu.SemaphoreType.DMA,
            pltpu.SemaphoreType.DMA,
            pltpu.SemaphoreType.DMA,
            pltpu.SemaphoreType.DMA,
        ],
    )
    def k(table_hbm, idx_hbm, out_hbm, idx_v, rows_a, rows_b,
          sga, sgb, swa, swb):
        wid = lax.axis_index("s") * 2 + lax.axis_index("c")
        base = wid * EPW

        pltpu.async_copy(idx_hbm.at[pl.ds(base, EPW)], idx_v, sga).wait()

        @pl.loop(0, 2 * NPAIR * CH, step=2 * CH)
        def _(c0):
            a = base + c0
            b = a + CH
            ga = pltpu.async_copy(table_hbm.at[idx_v.at[pl.ds(c0, CH)]],
                                  rows_a, sga)
            gb = pltpu.async_copy(table_hbm.at[idx_v.at[pl.ds(c0 + CH, CH)]],
                                  rows_b, sgb)
            ga.wait()
            wa = pltpu.async_copy(rows_a, out_hbm.at[pl.ds(a, CH)], swa)
            gb.wait()
            wb = pltpu.async_copy(rows_b, out_hbm.at[pl.ds(b, CH)], swb)
            wa.wait()
            wb.wait()

        tail = 2 * NPAIR * CH
        ga = pltpu.async_copy(table_hbm.at[idx_v.at[pl.ds(tail, CH)]],
                              rows_a, sga)
        ga.wait()
        pltpu.sync_copy(rows_a, out_hbm.at[pl.ds(base + tail, CH)])

    return k(table, idx)


def _residual_chain(x, wr, k, n):
    for _ in range(n):
        w1, b1, w2, b2 = (wr[k][...], wr[k + 1][...], wr[k + 2][...],
                          wr[k + 3][...])
        k += 4
        y = _ssp(x)
        y = _ssp(jnp.dot(y, w1, preferred_element_type=jnp.float32) + b1)
        x = x + jnp.dot(y, w2, preferred_element_type=jnp.float32) + b2
    return x, k


def _block_body(lo_ref, nw_ref, cut_ref, rbf_ref, gath_ref, idx_ref, wg_ref,
                x_ref, xi_ref, *rest, with_next):
    nw_weights = 23 + (4 if with_next else 0)
    wr = rest[:nw_weights]
    outs = rest[nw_weights:-1]
    acc = rest[-1]
    t = pl.program_id(0)

    @pl.when(t == 0)
    def _():
        acc[...] = jnp.zeros_like(acc)

    @pl.when(t < NT)
    def _():
        desc = (cut_ref[...] * rbf_ref[...]).astype(jnp.bfloat16)
        g = jnp.dot(desc, wg_ref[...],
                    preferred_element_type=jnp.float32).astype(jnp.bfloat16)
        xj = g * gath_ref[...].astype(jnp.bfloat16)
        idx = idx_ref[0]
        ts = jnp.minimum(t, NT - 1)
        lo = lo_ref[ts]
        nw = nw_ref[ts]

        def w_body(w, carry):
            start = lo + w * W
            base = jnp.minimum(start, NA - W)
            rel = idx - base
            inwin = (idx >= start) & (idx < start + W)
            oh = ((lax.broadcasted_iota(jnp.int32, (W, ET), 0) == rel)
                  & inwin).astype(jnp.bfloat16)
            part = lax.dot_general(oh, xj, (((1,), (0,)), ((), ())),
                                   preferred_element_type=jnp.float32)
            acc[pl.ds(base, W), :] += part
            return carry

        lax.fori_loop(0, nw, w_body, 0)

    @pl.when(t >= NT)
    def _():
        r = t - NT
        m = xi_ref[...] + acc[pl.ds(r * ROWS, ROWS), :]
        m, k = _residual_chain(m, wr, 0, NRI)
        m = _ssp(m)
        wo, bo, u = wr[k][...], wr[k + 1][...], wr[k + 2][...]
        k += 3
        x = u * x_ref[...] + jnp.dot(
            m, wo, preferred_element_type=jnp.float32) + bo
        x, k = _residual_chain(x, wr, k, NRF)
        outs[0][...] = x
        if with_next:
            wi, bi, wj, bj = (wr[k][...], wr[k + 1][...], wr[k + 2][...],
                              wr[k + 3][...])
            xa = _ssp(x)
            outs[1][...] = _ssp(
                jnp.dot(xa, wi, preferred_element_type=jnp.float32) + bi)
            outs[2][...] = _ssp(
                jnp.dot(xa, wj, preferred_element_type=jnp.float32) + bj)


def _block(lo_arr, nw_arr, cut2, rbfs, gath, idx3, wg, x, xi, weights,
           with_next):
    def emap(t, lo, nw):
        ts = jnp.minimum(t, NT - 1)
        return (ts, 0)

    def emap3(t, lo, nw):
        ts = jnp.minimum(t, NT - 1)
        return (ts, 0, 0)

    def rmap(t, lo, nw):
        return (jnp.maximum(t - NT, 0), 0)

    def cmap(t, lo, nw):
        return (0, 0)

    def w_spec(a):
        if a.ndim == 2 and a.shape == (D, D):
            return pl.BlockSpec((D, D), cmap)
        return pl.BlockSpec((1, D), cmap)

    n_out = 3 if with_next else 1
    grid_spec = pltpu.PrefetchScalarGridSpec(
        num_scalar_prefetch=2,
        grid=(NT + NRT,),
        in_specs=[
            pl.BlockSpec((ET, 1), emap),
            pl.BlockSpec((ET, NRBF), emap),
            pl.BlockSpec((ET, D), emap),
            pl.BlockSpec((1, 1, ET), emap3),
            pl.BlockSpec((NRBF, D), cmap),
            pl.BlockSpec((ROWS, D), rmap),
            pl.BlockSpec((ROWS, D), rmap),
        ] + [w_spec(a) for a in weights],
        out_specs=[pl.BlockSpec((ROWS, D), rmap)] * n_out,
        scratch_shapes=[pltpu.VMEM((NA, D), jnp.float32)],
    )
    out = pl.pallas_call(
        functools.partial(_block_body, with_next=with_next),
        grid_spec=grid_spec,
        out_shape=[jax.ShapeDtypeStruct((NA, D), jnp.float32)] * n_out,
    )(lo_arr, nw_arr, cut2, rbfs, gath, idx3, wg, x, xi, *weights)
    return out


def kernel(features, distances, cutoffs, rbfs, idx_i, idx_j, params):
    del distances
    cut2 = cutoffs.reshape(NE, 1)
    idx3 = idx_i.reshape(NT, 1, ET)
    lo_arr = (idx_i[::ET] // 8) * 8
    hi_arr = idx_i[ET - 1::ET]
    nw_arr = (hi_arr - lo_arr) // W + 1

    p = params

    def post_weights(b):
        ws = []
        for r in range(NRI):
            ws += [p["Wri1"][b][r], p["bri1"][b][r].reshape(1, D),
                   p["Wri2"][b][r], p["bri2"][b][r].reshape(1, D)]
        ws += [p["Wo"][b], p["bo"][b].reshape(1, D), p["u"][b].reshape(1, D)]
        for r in range(NRF):
            ws += [p["Wrf1"][b][r], p["brf1"][b][r].reshape(1, D),
                   p["Wrf2"][b][r], p["brf2"][b][r].reshape(1, D)]
        return ws

    x = features
    xi, xj_all = _pre(x, p["Wi"][0], p["bi"][0], p["Wj"][0], p["bj"][0])
    outs = []
    for b in range(NBLK):
        gath = _sc_gather(xj_all, idx_j)
        weights = post_weights(b)
        with_next = b < NBLK - 1
        if with_next:
            weights += [p["Wi"][b + 1], p["bi"][b + 1].reshape(1, D),
                        p["Wj"][b + 1], p["bj"][b + 1].reshape(1, D)]
        res = _block(lo_arr, nw_arr, cut2, rbfs, gath, idx3,
                     p["Wg"][b].astype(jnp.bfloat16), x, xi, weights,
                     with_next)
        if with_next:
            x, xi, xj_all = res
        else:
            x = res[0]
        outs.append(x)
    return tuple(outs)


# speedup vs baseline: 1.0486x; 1.0486x over previous
import functools

import jax
import jax.numpy as jnp
from jax import lax
from jax.experimental import pallas as pl
from jax.experimental.pallas import tpu as pltpu
from jax.experimental.pallas import tpu_sc as plsc

NA = 10000
NE = 320000
D = 128
NRBF = 16
NBLK = 5
NRI = 3
NRF = 2
LOG2 = 0.6931471805599453

ROWS = 1000
NRT = NA // ROWS
ET = 1280
NT = NE // ET
W = 128

NWORK = 32
EPW = NE // NWORK
CH = 400
NPAIR = 12


def _ssp(x):
    return jax.nn.softplus(x) - LOG2


def _pre_body(x_ref, wi_ref, bi_ref, wj_ref, bj_ref, xi_ref, xj_ref):
    xa = _ssp(x_ref[...])
    xi_ref[...] = _ssp(
        jnp.dot(xa, wi_ref[...], preferred_element_type=jnp.float32) + bi_ref[...])
    xj_ref[...] = _ssp(
        jnp.dot(xa, wj_ref[...], preferred_element_type=jnp.float32) + bj_ref[...])


def _pre(x, wi, bi, wj, bj):
    return pl.pallas_call(
        _pre_body,
        grid=(NRT,),
        in_specs=[
            pl.BlockSpec((ROWS, D), lambda t: (t, 0)),
            pl.BlockSpec((D, D), lambda t: (0, 0)),
            pl.BlockSpec((1, D), lambda t: (0, 0)),
            pl.BlockSpec((D, D), lambda t: (0, 0)),
            pl.BlockSpec((1, D), lambda t: (0, 0)),
        ],
        out_specs=[pl.BlockSpec((ROWS, D), lambda t: (t, 0)),
                   pl.BlockSpec((ROWS, D), lambda t: (t, 0))],
        out_shape=[jax.ShapeDtypeStruct((NA, D), jnp.float32)] * 2,
    )(x, wi, bi.reshape(1, D), wj, bj.reshape(1, D))


def _sc_gather(table, idx):
    mesh = plsc.VectorSubcoreMesh(core_axis_name="c", subcore_axis_name="s")

    @functools.partial(
        pl.kernel, mesh=mesh,
        out_type=jax.ShapeDtypeStruct((NE, D), jnp.float32),
        scratch_types=[
            pltpu.VMEM((EPW,), jnp.int32),
            pltpu.VMEM((CH, D), jnp.float32),
            pltpu.VMEM((CH, D), jnp.float32),
            pltpu.SemaphoreType.DMA,
            pltpu.SemaphoreType.DMA,
            pltpu.SemaphoreType.DMA,
            pltpu.SemaphoreType.DMA,
        ],
    )
    def k(table_hbm, idx_hbm, out_hbm, idx_v, rows_a, rows_b,
          sga, sgb, swa, swb):
        wid = lax.axis_index("s") * 2 + lax.axis_index("c")
        base = wid * EPW

        pltpu.async_copy(idx_hbm.at[pl.ds(base, EPW)], idx_v, sga).wait()

        @pl.loop(0, 2 * NPAIR * CH, step=2 * CH)
        def _(c0):
            a = base + c0
            b = a + CH
            ga = pltpu.async_copy(table_hbm.at[idx_v.at[pl.ds(c0, CH)]],
                                  rows_a, sga)
            gb = pltpu.async_copy(table_hbm.at[idx_v.at[pl.ds(c0 + CH, CH)]],
                                  rows_b, sgb)
            ga.wait()
            wa = pltpu.async_copy(rows_a, out_hbm.at[pl.ds(a, CH)], swa)
            gb.wait()
            wb = pltpu.async_copy(rows_b, out_hbm.at[pl.ds(b, CH)], swb)
            wa.wait()
            wb.wait()

        tail = 2 * NPAIR * CH
        ga = pltpu.async_copy(table_hbm.at[idx_v.at[pl.ds(tail, CH)]],
                              rows_a, sga)
        ga.wait()
        pltpu.sync_copy(rows_a, out_hbm.at[pl.ds(base + tail, CH)])

    return k(table, idx)


EG = 2560
NTG = NE // EG


def _gall_body(cut_ref, rbf_ref, wg_ref, *out_refs):
    desc = (cut_ref[...] * rbf_ref[...]).astype(jnp.bfloat16)
    gall = jnp.dot(desc, wg_ref[...],
                   preferred_element_type=jnp.float32).astype(jnp.bfloat16)
    for b in range(NBLK):
        out_refs[b][...] = gall[:, b * D:(b + 1) * D]


def _gall(cut2, rbfs, wg_all):
    return pl.pallas_call(
        _gall_body,
        grid=(NTG,),
        in_specs=[
            pl.BlockSpec((EG, 1), lambda t: (t, 0)),
            pl.BlockSpec((EG, NRBF), lambda t: (t, 0)),
            pl.BlockSpec((NRBF, NBLK * D), lambda t: (0, 0)),
        ],
        out_specs=[pl.BlockSpec((EG, D), lambda t: (t, 0))] * NBLK,
        out_shape=[jax.ShapeDtypeStruct((NE, D), jnp.bfloat16)] * NBLK,
    )(cut2, rbfs, wg_all)


def _residual_chain(x, wr, k, n):
    for _ in range(n):
        w1, b1, w2, b2 = (wr[k][...], wr[k + 1][...], wr[k + 2][...],
                          wr[k + 3][...])
        k += 4
        y = _ssp(x)
        y = _ssp(jnp.dot(y, w1, preferred_element_type=jnp.float32) + b1)
        x = x + jnp.dot(y, w2, preferred_element_type=jnp.float32) + b2
    return x, k


def _block_body(lo_ref, nw_ref, g_ref, gath_ref, idx_ref,
                x_ref, xi_ref, *rest, with_next):
    nw_weights = 23 + (4 if with_next else 0)
    wr = rest[:nw_weights]
    outs = rest[nw_weights:-1]
    acc = rest[-1]
    t = pl.program_id(0)

    @pl.when(t == 0)
    def _():
        acc[...] = jnp.zeros_like(acc)

    @pl.when(t < NT)
    def _():
        xj = g_ref[...] * gath_ref[...].astype(jnp.bfloat16)
        idx = idx_ref[0]
        ts = jnp.minimum(t, NT - 1)
        lo = lo_ref[ts]
        nw = nw_ref[ts]

        def w_body(w, carry):
            start = lo + w * W
            base = jnp.minimum(start, NA - W)
            rel = idx - base
            inwin = (idx >= start) & (idx < start + W)
            oh = ((lax.broadcasted_iota(jnp.int32, (W, ET), 0) == rel)
                  & inwin).astype(jnp.bfloat16)
            part = lax.dot_general(oh, xj, (((1,), (0,)), ((), ())),
                                   preferred_element_type=jnp.float32)
            acc[pl.ds(base, W), :] += part
            return carry

        lax.fori_loop(0, nw, w_body, 0)

    @pl.when(t >= NT)
    def _():
        r = t - NT
        m = xi_ref[...] + acc[pl.ds(r * ROWS, ROWS), :]
        m, k = _residual_chain(m, wr, 0, NRI)
        m = _ssp(m)
        wo, bo, u = wr[k][...], wr[k + 1][...], wr[k + 2][...]
        k += 3
        x = u * x_ref[...] + jnp.dot(
            m, wo, preferred_element_type=jnp.float32) + bo
        x, k = _residual_chain(x, wr, k, NRF)
        outs[0][...] = x
        if with_next:
            wi, bi, wj, bj = (wr[k][...], wr[k + 1][...], wr[k + 2][...],
                              wr[k + 3][...])
            xa = _ssp(x)
            outs[1][...] = _ssp(
                jnp.dot(xa, wi, preferred_element_type=jnp.float32) + bi)
            outs[2][...] = _ssp(
                jnp.dot(xa, wj, preferred_element_type=jnp.float32) + bj)


def _block(lo_arr, nw_arr, g, gath, idx3, x, xi, weights, with_next):
    def emap(t, lo, nw):
        ts = jnp.minimum(t, NT - 1)
        return (ts, 0)

    def emap3(t, lo, nw):
        ts = jnp.minimum(t, NT - 1)
        return (ts, 0, 0)

    def rmap(t, lo, nw):
        return (jnp.maximum(t - NT, 0), 0)

    def cmap(t, lo, nw):
        return (0, 0)

    def w_spec(a):
        if a.ndim == 2 and a.shape == (D, D):
            return pl.BlockSpec((D, D), cmap)
        return pl.BlockSpec((1, D), cmap)

    n_out = 3 if with_next else 1
    grid_spec = pltpu.PrefetchScalarGridSpec(
        num_scalar_prefetch=2,
        grid=(NT + NRT,),
        in_specs=[
            pl.BlockSpec((ET, D), emap),
            pl.BlockSpec((ET, D), emap),
            pl.BlockSpec((1, 1, ET), emap3),
            pl.BlockSpec((ROWS, D), rmap),
            pl.BlockSpec((ROWS, D), rmap),
        ] + [w_spec(a) for a in weights],
        out_specs=[pl.BlockSpec((ROWS, D), rmap)] * n_out,
        scratch_shapes=[pltpu.VMEM((NA, D), jnp.float32)],
    )
    out = pl.pallas_call(
        functools.partial(_block_body, with_next=with_next),
        grid_spec=grid_spec,
        out_shape=[jax.ShapeDtypeStruct((NA, D), jnp.float32)] * n_out,
    )(lo_arr, nw_arr, g, gath, idx3, x, xi, *weights)
    return out


def kernel(features, distances, cutoffs, rbfs, idx_i, idx_j, params):
    del distances
    cut2 = cutoffs.reshape(NE, 1)
    idx3 = idx_i.reshape(NT, 1, ET)
    lo_arr = (idx_i[::ET] // 8) * 8
    hi_arr = idx_i[ET - 1::ET]
    nw_arr = (hi_arr - lo_arr) // W + 1

    p = params

    def post_weights(b):
        ws = []
        for r in range(NRI):
            ws += [p["Wri1"][b][r], p["bri1"][b][r].reshape(1, D),
                   p["Wri2"][b][r], p["bri2"][b][r].reshape(1, D)]
        ws += [p["Wo"][b], p["bo"][b].reshape(1, D), p["u"][b].reshape(1, D)]
        for r in range(NRF):
            ws += [p["Wrf1"][b][r], p["brf1"][b][r].reshape(1, D),
                   p["Wrf2"][b][r], p["brf2"][b][r].reshape(1, D)]
        return ws

    wg_all = jnp.concatenate([p["Wg"][b] for b in range(NBLK)],
                             axis=1).astype(jnp.bfloat16)
    gs = _gall(cut2, rbfs, wg_all)

    x = features
    xi, xj_all = _pre(x, p["Wi"][0], p["bi"][0], p["Wj"][0], p["bj"][0])
    outs = []
    for b in range(NBLK):
        gath = _sc_gather(xj_all, idx_j)
        weights = post_weights(b)
        with_next = b < NBLK - 1
        if with_next:
            weights += [p["Wi"][b + 1], p["bi"][b + 1].reshape(1, D),
                        p["Wj"][b + 1], p["bj"][b + 1].reshape(1, D)]
        res = _block(lo_arr, nw_arr, gs[b], gath, idx3, x, xi, weights,
                     with_next)
        if with_next:
            x, xi, xj_all = res
        else:
            x = res[0]
        outs.append(x)
    return tuple(outs)


# speedup vs baseline: 1.1040x; 1.0528x over previous
import functools

import jax
import jax.numpy as jnp
from jax import lax
from jax.experimental import pallas as pl
from jax.experimental.pallas import tpu as pltpu
from jax.experimental.pallas import tpu_sc as plsc

NA = 10000
NE = 320000
D = 128
NRBF = 16
NBLK = 5
NRI = 3
NRF = 2
LOG2 = 0.6931471805599453

ROWS = 1000
NRT = NA // ROWS
ET = 1280
NT = NE // ET
W = 128

NWORK = 32
EPW = NE // NWORK
CH = 400
NPAIR = 12


def _ssp(x):
    return jax.nn.softplus(x) - LOG2


def _pre_body(x_ref, wi_ref, bi_ref, wj_ref, bj_ref, xi_ref, xj_ref):
    xa = _ssp(x_ref[...])
    xi_ref[...] = _ssp(
        jnp.dot(xa, wi_ref[...], preferred_element_type=jnp.float32) + bi_ref[...])
    xj_ref[...] = _ssp(
        jnp.dot(xa, wj_ref[...], preferred_element_type=jnp.float32) + bj_ref[...])


def _pre(x, wi, bi, wj, bj):
    return pl.pallas_call(
        _pre_body,
        grid=(NRT,),
        in_specs=[
            pl.BlockSpec((ROWS, D), lambda t: (t, 0)),
            pl.BlockSpec((D, D), lambda t: (0, 0)),
            pl.BlockSpec((1, D), lambda t: (0, 0)),
            pl.BlockSpec((D, D), lambda t: (0, 0)),
            pl.BlockSpec((1, D), lambda t: (0, 0)),
        ],
        out_specs=[pl.BlockSpec((ROWS, D), lambda t: (t, 0)),
                   pl.BlockSpec((ROWS, D), lambda t: (t, 0))],
        out_shape=[jax.ShapeDtypeStruct((NA, D), jnp.float32)] * 2,
    )(x, wi, bi.reshape(1, D), wj, bj.reshape(1, D))


def _sc_gather(table, idx):
    mesh = plsc.VectorSubcoreMesh(core_axis_name="c", subcore_axis_name="s")

    @functools.partial(
        pl.kernel, mesh=mesh,
        out_type=jax.ShapeDtypeStruct((NE, D), jnp.float32),
        scratch_types=[
            pltpu.VMEM((EPW,), jnp.int32),
            pltpu.VMEM((CH, D), jnp.float32),
            pltpu.VMEM((CH, D), jnp.float32),
            pltpu.SemaphoreType.DMA,
            pltpu.SemaphoreType.DMA,
            pltpu.SemaphoreType.DMA,
            pltpu.SemaphoreType.DMA,
        ],
    )
    def k(table_hbm, idx_hbm, out_hbm, idx_v, rows_a, rows_b,
          sga, sgb, swa, swb):
        wid = lax.axis_index("s") * 2 + lax.axis_index("c")
        base = wid * EPW

        pltpu.async_copy(idx_hbm.at[pl.ds(base, EPW)], idx_v, sga).wait()

        @pl.loop(0, 2 * NPAIR * CH, step=2 * CH)
        def _(c0):
            a = base + c0
            b = a + CH
            ga = pltpu.async_copy(table_hbm.at[idx_v.at[pl.ds(c0, CH)]],
                                  rows_a, sga)
            gb = pltpu.async_copy(table_hbm.at[idx_v.at[pl.ds(c0 + CH, CH)]],
                                  rows_b, sgb)
            ga.wait()
            wa = pltpu.async_copy(rows_a, out_hbm.at[pl.ds(a, CH)], swa)
            gb.wait()
            wb = pltpu.async_copy(rows_b, out_hbm.at[pl.ds(b, CH)], swb)
            wa.wait()
            wb.wait()

        tail = 2 * NPAIR * CH
        ga = pltpu.async_copy(table_hbm.at[idx_v.at[pl.ds(tail, CH)]],
                              rows_a, sga)
        ga.wait()
        pltpu.sync_copy(rows_a, out_hbm.at[pl.ds(base + tail, CH)])

    return k(table, idx)


EG = 2560
NTG = NE // EG


def _gall_body(rbf_ref, wg_ref, out_ref):
    desc = rbf_ref[...].astype(jnp.bfloat16)
    out_ref[...] = jnp.dot(desc, wg_ref[...],
                           preferred_element_type=jnp.float32).astype(
        jnp.bfloat16)


def _gall(rbfs, wg_all):
    return pl.pallas_call(
        _gall_body,
        grid=(NTG,),
        in_specs=[
            pl.BlockSpec((EG, NRBF), lambda t: (t, 0)),
            pl.BlockSpec((NRBF, NBLK * D), lambda t: (0, 0)),
        ],
        out_specs=pl.BlockSpec((EG, NBLK * D), lambda t: (t, 0)),
        out_shape=jax.ShapeDtypeStruct((NE, NBLK * D), jnp.bfloat16),
    )(rbfs, wg_all)


def _residual_chain(x, wr, k, n):
    for _ in range(n):
        w1, b1, w2, b2 = (wr[k][...], wr[k + 1][...], wr[k + 2][...],
                          wr[k + 3][...])
        k += 4
        y = _ssp(x)
        y = _ssp(jnp.dot(y, w1, preferred_element_type=jnp.float32) + b1)
        x = x + jnp.dot(y, w2, preferred_element_type=jnp.float32) + b2
    return x, k


def _block_body(lo_ref, nw_ref, g_ref, gath_ref, idx_ref, cut_ref,
                x_ref, xi_ref, *rest, with_next):
    nw_weights = 23 + (4 if with_next else 0)
    wr = rest[:nw_weights]
    outs = rest[nw_weights:-1]
    acc = rest[-1]
    t = pl.program_id(0)

    @pl.when(t == 0)
    def _():
        acc[...] = jnp.zeros_like(acc)

    @pl.when(t < NT)
    def _():
        xj = g_ref[...] * gath_ref[...].astype(jnp.bfloat16)
        idx = idx_ref[0]
        cutv = cut_ref[0].astype(jnp.bfloat16)
        ts = jnp.minimum(t, NT - 1)
        lo = lo_ref[ts]
        nw = nw_ref[ts]

        def w_body(w, carry):
            start = lo + w * W
            base = jnp.minimum(start, NA - W)
            rel = idx - base
            inwin = (idx >= start) & (idx < start + W)
            oh = ((lax.broadcasted_iota(jnp.int32, (W, ET), 0) == rel)
                  & inwin).astype(jnp.bfloat16) * cutv
            part = lax.dot_general(oh, xj, (((1,), (0,)), ((), ())),
                                   preferred_element_type=jnp.float32)
            acc[pl.ds(base, W), :] += part
            return carry

        lax.fori_loop(0, nw, w_body, 0)

    @pl.when(t >= NT)
    def _():
        r = t - NT
        m = xi_ref[...] + acc[pl.ds(r * ROWS, ROWS), :]
        m, k = _residual_chain(m, wr, 0, NRI)
        m = _ssp(m)
        wo, bo, u = wr[k][...], wr[k + 1][...], wr[k + 2][...]
        k += 3
        x = u * x_ref[...] + jnp.dot(
            m, wo, preferred_element_type=jnp.float32) + bo
        x, k = _residual_chain(x, wr, k, NRF)
        outs[0][...] = x
        if with_next:
            wi, bi, wj, bj = (wr[k][...], wr[k + 1][...], wr[k + 2][...],
                              wr[k + 3][...])
            xa = _ssp(x)
            outs[1][...] = _ssp(
                jnp.dot(xa, wi, preferred_element_type=jnp.float32) + bi)
            outs[2][...] = _ssp(
                jnp.dot(xa, wj, preferred_element_type=jnp.float32) + bj)


def _block(lo_arr, nw_arr, gall, gath, idx3, cut3, x, xi, weights, with_next,
           b):
    def emap(t, lo, nw):
        ts = jnp.minimum(t, NT - 1)
        return (ts, 0)

    def gmap(t, lo, nw):
        ts = jnp.minimum(t, NT - 1)
        return (ts, b)

    def emap3(t, lo, nw):
        ts = jnp.minimum(t, NT - 1)
        return (ts, 0, 0)

    def rmap(t, lo, nw):
        return (jnp.maximum(t - NT, 0), 0)

    def cmap(t, lo, nw):
        return (0, 0)

    def w_spec(a):
        if a.ndim == 2 and a.shape == (D, D):
            return pl.BlockSpec((D, D), cmap)
        return pl.BlockSpec((1, D), cmap)

    n_out = 3 if with_next else 1
    grid_spec = pltpu.PrefetchScalarGridSpec(
        num_scalar_prefetch=2,
        grid=(NT + NRT,),
        in_specs=[
            pl.BlockSpec((ET, D), gmap),
            pl.BlockSpec((ET, D), emap),
            pl.BlockSpec((1, 1, ET), emap3),
            pl.BlockSpec((1, 1, ET), emap3),
            pl.BlockSpec((ROWS, D), rmap),
            pl.BlockSpec((ROWS, D), rmap),
        ] + [w_spec(a) for a in weights],
        out_specs=[pl.BlockSpec((ROWS, D), rmap)] * n_out,
        scratch_shapes=[pltpu.VMEM((NA, D), jnp.float32)],
    )
    out = pl.pallas_call(
        functools.partial(_block_body, with_next=with_next),
        grid_spec=grid_spec,
        out_shape=[jax.ShapeDtypeStruct((NA, D), jnp.float32)] * n_out,
    )(lo_arr, nw_arr, gall, gath, idx3, cut3, x, xi, *weights)
    return out


def kernel(features, distances, cutoffs, rbfs, idx_i, idx_j, params):
    del distances
    cut3 = cutoffs.reshape(NT, 1, ET)
    idx3 = idx_i.reshape(NT, 1, ET)
    lo_arr = (idx_i[::ET] // 8) * 8
    hi_arr = idx_i[ET - 1::ET]
    nw_arr = (hi_arr - lo_arr) // W + 1

    p = params

    def post_weights(b):
        ws = []
        for r in range(NRI):
            ws += [p["Wri1"][b][r], p["bri1"][b][r].reshape(1, D),
                   p["Wri2"][b][r], p["bri2"][b][r].reshape(1, D)]
        ws += [p["Wo"][b], p["bo"][b].reshape(1, D), p["u"][b].reshape(1, D)]
        for r in range(NRF):
            ws += [p["Wrf1"][b][r], p["brf1"][b][r].reshape(1, D),
                   p["Wrf2"][b][r], p["brf2"][b][r].reshape(1, D)]
        return ws

    wg_all = jnp.concatenate([p["Wg"][b] for b in range(NBLK)],
                             axis=1).astype(jnp.bfloat16)
    gall = _gall(rbfs, wg_all)

    x = features
    xi, xj_all = _pre(x, p["Wi"][0], p["bi"][0], p["Wj"][0], p["bj"][0])
    outs = []
    for b in range(NBLK):
        gath = _sc_gather(xj_all, idx_j)
        weights = post_weights(b)
        with_next = b < NBLK - 1
        if with_next:
            weights += [p["Wi"][b + 1], p["bi"][b + 1].reshape(1, D),
                        p["Wj"][b + 1], p["bj"][b + 1].reshape(1, D)]
        res = _block(lo_arr, nw_arr, gall, gath, idx3, cut3, x, xi, weights,
                     with_next, b)
        if with_next:
            x, xi, xj_all = res
        else:
            x = res[0]
        outs.append(x)
    return tuple(outs)


# speedup vs baseline: 1.1141x; 1.0091x over previous
import functools

import jax
import jax.numpy as jnp
from jax import lax
from jax.experimental import pallas as pl
from jax.experimental.pallas import tpu as pltpu
from jax.experimental.pallas import tpu_sc as plsc

NA = 10000
NE = 320000
D = 128
NRBF = 16
NBLK = 5
NRI = 3
NRF = 2
LOG2 = 0.6931471805599453

ROWS = 1000
NRT = NA // ROWS
ET = 1280
NT = NE // ET
W = 64

NWORK = 32
EPW = NE // NWORK
CH = 400
NPAIR = 12


def _ssp(x):
    return jax.nn.softplus(x) - LOG2


def _pre_body(x_ref, wi_ref, bi_ref, wj_ref, bj_ref, xi_ref, xj_ref):
    xa = _ssp(x_ref[...])
    xi_ref[...] = _ssp(
        jnp.dot(xa, wi_ref[...], preferred_element_type=jnp.float32) + bi_ref[...])
    xj_ref[...] = _ssp(
        jnp.dot(xa, wj_ref[...], preferred_element_type=jnp.float32) + bj_ref[...])


def _pre(x, wi, bi, wj, bj):
    return pl.pallas_call(
        _pre_body,
        grid=(NRT,),
        in_specs=[
            pl.BlockSpec((ROWS, D), lambda t: (t, 0)),
            pl.BlockSpec((D, D), lambda t: (0, 0)),
            pl.BlockSpec((1, D), lambda t: (0, 0)),
            pl.BlockSpec((D, D), lambda t: (0, 0)),
            pl.BlockSpec((1, D), lambda t: (0, 0)),
        ],
        out_specs=[pl.BlockSpec((ROWS, D), lambda t: (t, 0)),
                   pl.BlockSpec((ROWS, D), lambda t: (t, 0))],
        out_shape=[jax.ShapeDtypeStruct((NA, D), jnp.float32)] * 2,
    )(x, wi, bi.reshape(1, D), wj, bj.reshape(1, D))


def _sc_gather(table, idx):
    mesh = plsc.VectorSubcoreMesh(core_axis_name="c", subcore_axis_name="s")

    @functools.partial(
        pl.kernel, mesh=mesh,
        out_type=jax.ShapeDtypeStruct((NE, D), jnp.float32),
        scratch_types=[
            pltpu.VMEM((EPW,), jnp.int32),
            pltpu.VMEM((CH, D), jnp.float32),
            pltpu.VMEM((CH, D), jnp.float32),
            pltpu.SemaphoreType.DMA,
            pltpu.SemaphoreType.DMA,
            pltpu.SemaphoreType.DMA,
            pltpu.SemaphoreType.DMA,
        ],
    )
    def k(table_hbm, idx_hbm, out_hbm, idx_v, rows_a, rows_b,
          sga, sgb, swa, swb):
        wid = lax.axis_index("s") * 2 + lax.axis_index("c")
        base = wid * EPW

        pltpu.async_copy(idx_hbm.at[pl.ds(base, EPW)], idx_v, sga).wait()

        @pl.loop(0, 2 * NPAIR * CH, step=2 * CH)
        def _(c0):
            a = base + c0
            b = a + CH
            ga = pltpu.async_copy(table_hbm.at[idx_v.at[pl.ds(c0, CH)]],
                                  rows_a, sga)
            gb = pltpu.async_copy(table_hbm.at[idx_v.at[pl.ds(c0 + CH, CH)]],
                                  rows_b, sgb)
            ga.wait()
            wa = pltpu.async_copy(rows_a, out_hbm.at[pl.ds(a, CH)], swa)
            gb.wait()
            wb = pltpu.async_copy(rows_b, out_hbm.at[pl.ds(b, CH)], swb)
            wa.wait()
            wb.wait()

        tail = 2 * NPAIR * CH
        ga = pltpu.async_copy(table_hbm.at[idx_v.at[pl.ds(tail, CH)]],
                              rows_a, sga)
        ga.wait()
        pltpu.sync_copy(rows_a, out_hbm.at[pl.ds(base + tail, CH)])

    return k(table, idx)


EG = 2560
NTG = NE // EG


def _gall_body(rbf_ref, wg_ref, out_ref):
    desc = rbf_ref[...].astype(jnp.bfloat16)
    out_ref[...] = jnp.dot(desc, wg_ref[...],
                           preferred_element_type=jnp.float32).astype(
        jnp.bfloat16)


def _gall(rbfs, wg_all):
    return pl.pallas_call(
        _gall_body,
        grid=(NTG,),
        in_specs=[
            pl.BlockSpec((EG, NRBF), lambda t: (t, 0)),
            pl.BlockSpec((NRBF, NBLK * D), lambda t: (0, 0)),
        ],
        out_specs=pl.BlockSpec((EG, NBLK * D), lambda t: (t, 0)),
        out_shape=jax.ShapeDtypeStruct((NE, NBLK * D), jnp.bfloat16),
    )(rbfs, wg_all)


def _residual_chain(x, wr, k, n):
    for _ in range(n):
        w1, b1, w2, b2 = (wr[k][...], wr[k + 1][...], wr[k + 2][...],
                          wr[k + 3][...])
        k += 4
        y = _ssp(x)
        y = _ssp(jnp.dot(y, w1, preferred_element_type=jnp.float32) + b1)
        x = x + jnp.dot(y, w2, preferred_element_type=jnp.float32) + b2
    return x, k


def _block_body(lo_ref, nw_ref, g_ref, gath_ref, idx_ref, cut_ref,
                x_ref, xi_ref, *rest, with_next):
    nw_weights = 23 + (4 if with_next else 0)
    wr = rest[:nw_weights]
    outs = rest[nw_weights:-1]
    acc = rest[-1]
    t = pl.program_id(0)

    @pl.when(t == 0)
    def _():
        acc[...] = jnp.zeros_like(acc)

    @pl.when(t < NT)
    def _():
        xj = g_ref[...] * gath_ref[...].astype(jnp.bfloat16)
        idx = idx_ref[0]
        cutv = cut_ref[0].astype(jnp.bfloat16)
        ts = jnp.minimum(t, NT - 1)
        lo = lo_ref[ts]
        nw = nw_ref[ts]

        def w_body(w, carry):
            start = lo + w * W
            base = jnp.minimum(start, NA - W)
            rel = idx - base
            inwin = (idx >= start) & (idx < start + W)
            oh = ((lax.broadcasted_iota(jnp.int32, (W, ET), 0) == rel)
                  & inwin).astype(jnp.bfloat16) * cutv
            part = lax.dot_general(oh, xj, (((1,), (0,)), ((), ())),
                                   preferred_element_type=jnp.float32)
            acc[pl.ds(base, W), :] += part
            return carry

        lax.fori_loop(0, nw, w_body, 0)

    @pl.when(t >= NT)
    def _():
        r = t - NT
        m = xi_ref[...] + acc[pl.ds(r * ROWS, ROWS), :]
        m, k = _residual_chain(m, wr, 0, NRI)
        m = _ssp(m)
        wo, bo, u = wr[k][...], wr[k + 1][...], wr[k + 2][...]
        k += 3
        x = u * x_ref[...] + jnp.dot(
            m, wo, preferred_element_type=jnp.float32) + bo
        x, k = _residual_chain(x, wr, k, NRF)
        outs[0][...] = x
        if with_next:
            wi, bi, wj, bj = (wr[k][...], wr[k + 1][...], wr[k + 2][...],
                              wr[k + 3][...])
            xa = _ssp(x)
            outs[1][...] = _ssp(
                jnp.dot(xa, wi, preferred_element_type=jnp.float32) + bi)
            outs[2][...] = _ssp(
                jnp.dot(xa, wj, preferred_element_type=jnp.float32) + bj)


def _block(lo_arr, nw_arr, gall, gath, idx3, cut3, x, xi, weights, with_next,
           b):
    def emap(t, lo, nw):
        ts = jnp.minimum(t, NT - 1)
        return (ts, 0)

    def gmap(t, lo, nw):
        ts = jnp.minimum(t, NT - 1)
        return (ts, b)

    def emap3(t, lo, nw):
        ts = jnp.minimum(t, NT - 1)
        return (ts, 0, 0)

    def rmap(t, lo, nw):
        return (jnp.maximum(t - NT, 0), 0)

    def cmap(t, lo, nw):
        return (0, 0)

    def w_spec(a):
        if a.ndim == 2 and a.shape == (D, D):
            return pl.BlockSpec((D, D), cmap)
        return pl.BlockSpec((1, D), cmap)

    n_out = 3 if with_next else 1
    grid_spec = pltpu.PrefetchScalarGridSpec(
        num_scalar_prefetch=2,
        grid=(NT + NRT,),
        in_specs=[
            pl.BlockSpec((ET, D), gmap),
            pl.BlockSpec((ET, D), emap),
            pl.BlockSpec((1, 1, ET), emap3),
            pl.BlockSpec((1, 1, ET), emap3),
            pl.BlockSpec((ROWS, D), rmap),
            pl.BlockSpec((ROWS, D), rmap),
        ] + [w_spec(a) for a in weights],
        out_specs=[pl.BlockSpec((ROWS, D), rmap)] * n_out,
        scratch_shapes=[pltpu.VMEM((NA, D), jnp.float32)],
    )
    out = pl.pallas_call(
        functools.partial(_block_body, with_next=with_next),
        grid_spec=grid_spec,
        out_shape=[jax.ShapeDtypeStruct((NA, D), jnp.float32)] * n_out,
    )(lo_arr, nw_arr, gall, gath, idx3, cut3, x, xi, *weights)
    return out


def kernel(features, distances, cutoffs, rbfs, idx_i, idx_j, params):
    del distances
    cut3 = cutoffs.reshape(NT, 1, ET)
    idx3 = idx_i.reshape(NT, 1, ET)
    lo_arr = (idx_i[::ET] // 8) * 8
    hi_arr = idx_i[ET - 1::ET]
    nw_arr = (hi_arr - lo_arr) // W + 1

    p = params

    def post_weights(b):
        ws = []
        for r in range(NRI):
            ws += [p["Wri1"][b][r], p["bri1"][b][r].reshape(1, D),
                   p["Wri2"][b][r], p["bri2"][b][r].reshape(1, D)]
        ws += [p["Wo"][b], p["bo"][b].reshape(1, D), p["u"][b].reshape(1, D)]
        for r in range(NRF):
            ws += [p["Wrf1"][b][r], p["brf1"][b][r].reshape(1, D),
                   p["Wrf2"][b][r], p["brf2"][b][r].reshape(1, D)]
        return ws

    wg_all = jnp.concatenate([p["Wg"][b] for b in range(NBLK)],
                             axis=1).astype(jnp.bfloat16)
    gall = _gall(rbfs, wg_all)

    x = features
    xi, xj_all = _pre(x, p["Wi"][0], p["bi"][0], p["Wj"][0], p["bj"][0])
    outs = []
    for b in range(NBLK):
        gath = _sc_gather(xj_all, idx_j)
        weights = post_weights(b)
        with_next = b < NBLK - 1
        if with_next:
            weights += [p["Wi"][b + 1], p["bi"][b + 1].reshape(1, D),
                        p["Wj"][b + 1], p["bj"][b + 1].reshape(1, D)]
        res = _block(lo_arr, nw_arr, gall, gath, idx3, cut3, x, xi, weights,
                     with_next, b)
        if with_next:
            x, xi, xj_all = res
        else:
            x = res[0]
        outs.append(x)
    return tuple(outs)
